# baseline (speedup 1.0000x reference)
import functools

import jax
import jax.numpy as jnp
from jax import lax
from jax.experimental import pallas as pl
from jax.experimental.pallas import tpu as pltpu
from jax.experimental.pallas import tpu_sc as plsc

_ROWS = 16384
_COLS = 100
_NC = 2
_NS = 16
_NW = _NC * _NS
_RPW = _ROWS // _NW


def _make_sc_copy():
    mesh = plsc.VectorSubcoreMesh(core_axis_name="c", subcore_axis_name="s")

    @functools.partial(
        pl.kernel,
        mesh=mesh,
        out_type=jax.ShapeDtypeStruct((_ROWS, _COLS), jnp.float32),
        scratch_types=[pltpu.VMEM((_RPW, _COLS), jnp.float32)],
    )
    def sc_copy(in_hbm, out_hbm, stage_v):
        wid = lax.axis_index("s") * _NC + lax.axis_index("c")
        base = wid * _RPW
        pltpu.sync_copy(in_hbm.at[pl.ds(base, _RPW)], stage_v)
        pltpu.sync_copy(stage_v, out_hbm.at[pl.ds(base, _RPW)])

    return sc_copy


_sc_copy = _make_sc_copy()


def kernel(embeddings, table_event_type, table_entity_id, table_source_id,
           emb_linear_W, emb_linear_b, ln_gamma, ln_beta):
    del table_event_type, table_entity_id, table_source_id
    del emb_linear_W, emb_linear_b, ln_gamma, ln_beta
    return _sc_copy(embeddings)


# speedup vs baseline: 1.4795x; 1.4795x over previous
import jax
from jax.experimental import pallas as pl
from jax.experimental.pallas import tpu as pltpu


def _copy_kernel(in_ref, out_ref):
    out_ref[...] = in_ref[...]


def kernel(embeddings, table_event_type, table_entity_id, table_source_id,
           emb_linear_W, emb_linear_b, ln_gamma, ln_beta):
    del table_event_type, table_entity_id, table_source_id
    del emb_linear_W, emb_linear_b, ln_gamma, ln_beta
    rows, cols = embeddings.shape
    grid = 8
    return pl.pallas_call(
        _copy_kernel,
        out_shape=jax.ShapeDtypeStruct(embeddings.shape, embeddings.dtype),
        grid=(grid,),
        in_specs=[pl.BlockSpec((rows // grid, cols), lambda i: (i, 0))],
        out_specs=pl.BlockSpec((rows // grid, cols), lambda i: (i, 0)),
        compiler_params=pltpu.CompilerParams(
            dimension_semantics=("parallel",),
        ),
    )(embeddings)


# speedup vs baseline: 1.6840x; 1.1382x over previous
import jax
from jax.experimental import pallas as pl
from jax.experimental.pallas import tpu as pltpu

_ROWS = 16384
_COLS = 100
_CHUNKS = 8
_RPC = _ROWS // _CHUNKS


def _copy_kernel(in_hbm, out_hbm, stage, in_sems, out_sems):
    for i in range(_CHUNKS):
        pltpu.async_copy(
            in_hbm.at[pl.ds(i * _RPC, _RPC), :],
            stage.at[pl.ds(i * _RPC, _RPC), :],
            in_sems.at[i],
            priority=i % 2,
        )
    for i in range(_CHUNKS):
        pltpu.make_async_copy(
            in_hbm.at[pl.ds(i * _RPC, _RPC), :],
            stage.at[pl.ds(i * _RPC, _RPC), :],
            in_sems.at[i],
        ).wait()
        pltpu.async_copy(
            stage.at[pl.ds(i * _RPC, _RPC), :],
            out_hbm.at[pl.ds(i * _RPC, _RPC), :],
            out_sems.at[i],
            priority=i % 2,
        )
    for i in range(_CHUNKS):
        pltpu.make_async_copy(
            stage.at[pl.ds(i * _RPC, _RPC), :],
            out_hbm.at[pl.ds(i * _RPC, _RPC), :],
            out_sems.at[i],
        ).wait()


def kernel(embeddings, table_event_type, table_entity_id, table_source_id,
           emb_linear_W, emb_linear_b, ln_gamma, ln_beta):
    del table_event_type, table_entity_id, table_source_id
    del emb_linear_W, emb_linear_b, ln_gamma, ln_beta
    return pl.pallas_call(
        _copy_kernel,
        out_shape=jax.ShapeDtypeStruct(embeddings.shape, embeddings.dtype),
        in_specs=[pl.BlockSpec(memory_space=pl.ANY)],
        out_specs=pl.BlockSpec(memory_space=pl.ANY),
        scratch_shapes=[
            pltpu.VMEM((_ROWS, _COLS), embeddings.dtype),
            pltpu.SemaphoreType.DMA((_CHUNKS,)),
            pltpu.SemaphoreType.DMA((_CHUNKS,)),
        ],
    )(embeddings)
